# baseline (speedup 1.0000x reference)
import functools

import jax
import jax.numpy as jnp
import numpy as np
from jax import lax
from jax.experimental import pallas as pl
from jax.experimental.pallas import tpu as pltpu
from jax.experimental.pallas import tpu_sc as plsc

_LANES = 16


def _positional_encoding_np(d_model, seq_len=1000):
    position = np.arange(seq_len, dtype=np.float32).reshape(-1, 1)
    i = np.arange(d_model) // 2
    exp_term = (2.0 * i.astype(np.float32) / d_model).astype(np.float32)
    div_term = np.power(np.float32(10000.0), exp_term).reshape(1, -1)
    pe = (position / div_term).astype(np.float32)
    pe[:, 0::2] = np.sin(pe[:, 0::2])
    pe[:, 1::2] = np.cos(pe[:, 1::2])
    return pe


@functools.partial(jax.jit, static_argnames=("B", "S", "R", "D"))
def _run(data_flat, idx_flat, pos, global_token, *, B, S, R, D):
    info = plsc.get_sparse_core_info()
    NC, NS = info.num_cores, info.num_subcores
    NW = NC * NS
    N = B * R
    assert N % NW == 0
    RW = N // NW
    assert R % RW == 0
    CH = _LANES
    NCH = RW // CH
    NB = 3
    assert NCH >= NB and D % _LANES == 0
    assert B % 8 == 0
    NV = D // _LANES
    NV_SHIFT = NV.bit_length() - 1
    assert (1 << NV_SHIFT) == NV

    mesh = plsc.VectorSubcoreMesh(core_axis_name="c", subcore_axis_name="s")

    def body(data_hbm, idx_hbm, pos_hbm, gt_hbm, out_hbm,
             idx_v,
             didx0, didx1, didx2,
             oidx0, oidx1, oidx2,
             rows0, rows1, rows2, pos0, pos1,
             ga_v, gb_v, pos_sh,
             sd0, sd1, sd2, sp0, sp1, so0, so1, so2, sg):
        didx = (didx0, didx1, didx2)
        oidx = (oidx0, oidx1, oidx2)
        rows = (rows0, rows1, rows2)
        posb = (pos0, pos1)
        sd = (sd0, sd1, sd2)
        sp = (sp0, sp1)
        so = (so0, so1, so2)

        cid = lax.axis_index("c")
        sid = lax.axis_index("s")
        wid = sid * NC + cid
        base = wid * RW
        b = base // R
        r0 = base - b * R

        PR = (S + 1) // NS
        pltpu.sync_copy(pos_hbm.at[pl.ds(sid * PR, PR)],
                        pos_sh.at[pl.ds(sid * PR, PR)])

        @pl.when(sid == 0)
        def _():
            pltpu.sync_copy(pos_hbm.at[pl.ds(NS * PR, (S + 1) - NS * PR)],
                            pos_sh.at[pl.ds(NS * PR, (S + 1) - NS * PR)])

        pltpu.sync_copy(idx_hbm.at[pl.ds(base, RW)], idx_v)
        plsc.subcore_barrier()

        def set_indices(pr, pp, ci):
            off = ci * CH
            v = idx_v[pl.ds(off, CH)]
            didx[pr][...] = v + b * S
            oidx[pr][...] = (lax.iota(jnp.int32, _LANES)
                             + (r0 + off + 1)) * B + b

        def issue_gathers(pr, pp, ci):
            pltpu.async_copy(data_hbm.at[didx[pr]], rows[pr], sd[pr])
            v16 = idx_v[pl.ds(ci * CH, CH)] + 1
            for r in range(CH):
                pltpu.async_copy(pos_sh.at[pl.ds(v16[r], 1)],
                                 posb[pp].at[pl.ds(r, 1)], sp[pp])

        @pl.when(wid == 0)
        def _():
            pltpu.sync_copy(pos_hbm.at[pl.ds(0, 1)], gb_v)
            pltpu.sync_copy(gt_hbm, ga_v.at[pl.ds(0, 1)])

            @plsc.parallel_loop(0, NV, unroll=8)
            def _(j):
                sl = pl.ds(j * _LANES, _LANES)
                ga_v[0, sl] = ga_v[0, sl] + gb_v[0, sl]

            @plsc.parallel_loop(0, 7 * NV, unroll=8)
            def _(i):
                r = 1 + (i >> NV_SHIFT)
                sl = pl.ds((i & (NV - 1)) * _LANES, _LANES)
                ga_v[r, sl] = ga_v[0, sl]

            for g in range(B // 8):
                pltpu.async_copy(ga_v, out_hbm.at[pl.ds(g * 8, 8)], sg)

        set_indices(0, 0, 0)
        issue_gathers(0, 0, 0)
        set_indices(1, 1, 1)
        issue_gathers(1, 1, 1)

        for ci in range(NCH):
            p = ci % NB
            pp = ci % 2
            pltpu.make_async_copy(data_hbm.at[didx[p]], rows[p], sd[p]).wait()
            for r in range(CH):
                pltpu.make_async_copy(pos_sh.at[pl.ds(0, 1)],
                                      posb[pp].at[pl.ds(r, 1)],
                                      sp[pp]).wait()

            @plsc.parallel_loop(0, CH * NV, unroll=8)
            def _(i, p=p, pp=pp):
                r = i >> NV_SHIFT
                sl = pl.ds((i & (NV - 1)) * _LANES, _LANES)
                rows[p][r, sl] = rows[p][r, sl] + posb[pp][r, sl]

            pltpu.async_copy(rows[p], out_hbm.at[oidx[p]], so[p])

            nxt = ci + NB - 1
            if nxt < NCH:
                q = nxt % NB
                if nxt >= NB:
                    pltpu.make_async_copy(
                        rows[q], out_hbm.at[oidx[q]], so[q]).wait()
                set_indices(q, nxt % 2, nxt)
                issue_gathers(q, nxt % 2, nxt)

        for m in range(NCH - NB, NCH):
            pltpu.make_async_copy(
                rows[m % NB], out_hbm.at[oidx[m % NB]], so[m % NB]).wait()

        @pl.when(wid == 0)
        def _():
            for g in range(B // 8):
                pltpu.make_async_copy(
                    ga_v, out_hbm.at[pl.ds(g * 8, 8)], sg).wait()

    out_t_flat = pl.kernel(
        body,
        out_type=jax.ShapeDtypeStruct(((R + 1) * B, D), jnp.float32),
        mesh=mesh,
        scratch_types=[
            pltpu.VMEM((RW,), jnp.int32),
            pltpu.VMEM((CH,), jnp.int32),
            pltpu.VMEM((CH,), jnp.int32),
            pltpu.VMEM((CH,), jnp.int32),
            pltpu.VMEM((CH,), jnp.int32),
            pltpu.VMEM((CH,), jnp.int32),
            pltpu.VMEM((CH,), jnp.int32),
            pltpu.VMEM((CH, D), jnp.float32),
            pltpu.VMEM((CH, D), jnp.float32),
            pltpu.VMEM((CH, D), jnp.float32),
            pltpu.VMEM((CH, D), jnp.float32),
            pltpu.VMEM((CH, D), jnp.float32),
            pltpu.VMEM((8, D), jnp.float32),
            pltpu.VMEM((1, D), jnp.float32),
            pltpu.VMEM_SHARED((S + 1, D), jnp.float32),
            pltpu.SemaphoreType.DMA,
            pltpu.SemaphoreType.DMA,
            pltpu.SemaphoreType.DMA,
            pltpu.SemaphoreType.DMA,
            pltpu.SemaphoreType.DMA,
            pltpu.SemaphoreType.DMA,
            pltpu.SemaphoreType.DMA,
            pltpu.SemaphoreType.DMA,
            pltpu.SemaphoreType.DMA,
        ],
    )(data_flat, idx_flat, pos, global_token)
    return out_t_flat


def _ridx_tc_kernel(idx_ref, out_ref):
    b = pl.program_id(0)
    row = idx_ref[b, :]
    out_ref[...] = jnp.broadcast_to(row[None, :, None], out_ref.shape)


@functools.partial(jax.jit, static_argnames=("D",))
def _ridx_run(remain_idx, *, D):
    B, R = remain_idx.shape
    return pl.pallas_call(
        _ridx_tc_kernel,
        grid=(B,),
        in_specs=[pl.BlockSpec((B, R), lambda b: (0, 0))],
        out_specs=pl.BlockSpec((1, R, D), lambda b: (b, 0, 0)),
        out_shape=jax.ShapeDtypeStruct((B, R, D), jnp.int32),
    )(remain_idx)


def kernel(data, remain_idx, global_token):
    B, S, D = data.shape
    R = remain_idx.shape[1]
    pos = jnp.asarray(_positional_encoding_np(D)[: S + 1, :])
    out_t_flat = _run(
        data.reshape(B * S, D),
        remain_idx.reshape(B * R),
        pos,
        global_token,
        B=B, S=S, R=R, D=D,
    )
    ridx = _ridx_run(remain_idx, D=D)
    out = out_t_flat.reshape(R + 1, B, D).transpose(1, 0, 2)
    return (out, ridx)


# speedup vs baseline: 1.0513x; 1.0513x over previous
import functools

import jax
import jax.numpy as jnp
import numpy as np
from jax import lax
from jax.experimental import pallas as pl
from jax.experimental.pallas import tpu as pltpu
from jax.experimental.pallas import tpu_sc as plsc

_LANES = 16


def _positional_encoding_np(d_model, seq_len=1000):
    position = np.arange(seq_len, dtype=np.float32).reshape(-1, 1)
    i = np.arange(d_model) // 2
    exp_term = (2.0 * i.astype(np.float32) / d_model).astype(np.float32)
    div_term = np.power(np.float32(10000.0), exp_term).reshape(1, -1)
    pe = (position / div_term).astype(np.float32)
    pe[:, 0::2] = np.sin(pe[:, 0::2])
    pe[:, 1::2] = np.cos(pe[:, 1::2])
    return pe


@functools.partial(jax.jit, static_argnames=("B", "S", "R", "D"))
def _run(data_flat, idx_flat, pos, global_token, *, B, S, R, D):
    info = plsc.get_sparse_core_info()
    NC, NS = info.num_cores, info.num_subcores
    NW = NC * NS
    N = B * R
    assert N % NW == 0
    RW = N // NW
    assert R % RW == 0
    CH = _LANES
    NCH = RW // CH
    NB = 3
    assert NCH >= NB and D % _LANES == 0
    assert B % 8 == 0
    NV = D // _LANES
    NV_SHIFT = NV.bit_length() - 1
    assert (1 << NV_SHIFT) == NV

    mesh = plsc.VectorSubcoreMesh(core_axis_name="c", subcore_axis_name="s")

    def body(data_hbm, idx_hbm, pos_hbm, gt_hbm, out_hbm,
             idx_v,
             didx0, didx1, didx2, pidx0, pidx1, pidx2,
             oidx0, oidx1, oidx2,
             rows0, rows1, rows2, pos0, pos1, pos2,
             ga_v, gb_v,
             sd0, sd1, sd2, sp0, sp1, sp2, so0, so1, so2, sg):
        didx = (didx0, didx1, didx2)
        pidx = (pidx0, pidx1, pidx2)
        oidx = (oidx0, oidx1, oidx2)
        rows = (rows0, rows1, rows2)
        posb = (pos0, pos1, pos2)
        sd = (sd0, sd1, sd2)
        sp = (sp0, sp1, sp2)
        so = (so0, so1, so2)

        cid = lax.axis_index("c")
        sid = lax.axis_index("s")
        wid = sid * NC + cid
        base = wid * RW
        b = base // R
        r0 = base - b * R

        pltpu.sync_copy(idx_hbm.at[pl.ds(base, RW)], idx_v)

        def set_indices(p, ci):
            off = ci * CH
            v = idx_v[pl.ds(off, CH)]
            didx[p][...] = v + b * S
            pidx[p][...] = v + 1
            oidx[p][...] = (lax.iota(jnp.int32, _LANES)
                            + (r0 + off + 1)) * B + b

        def issue_gathers(p):
            pltpu.async_copy(data_hbm.at[didx[p]], rows[p], sd[p])
            pltpu.async_copy(pos_hbm.at[pidx[p]], posb[p], sp[p])

        @pl.when(wid == 0)
        def _():
            pltpu.sync_copy(pos_hbm.at[pl.ds(0, 1)], gb_v)
            pltpu.sync_copy(gt_hbm, ga_v.at[pl.ds(0, 1)])

            @plsc.parallel_loop(0, NV, unroll=8)
            def _(j):
                sl = pl.ds(j * _LANES, _LANES)
                ga_v[0, sl] = ga_v[0, sl] + gb_v[0, sl]

            @plsc.parallel_loop(0, (B - 1) * NV, unroll=8)
            def _(i):
                r = 1 + (i >> NV_SHIFT)
                sl = pl.ds((i & (NV - 1)) * _LANES, _LANES)
                ga_v[r, sl] = ga_v[0, sl]
            pltpu.async_copy(ga_v, out_hbm.at[pl.ds(0, B)], sg)

        set_indices(0, 0)
        issue_gathers(0)
        set_indices(1, 1)
        issue_gathers(1)

        for ci in range(NCH):
            p = ci % NB
            pltpu.make_async_copy(data_hbm.at[didx[p]], rows[p], sd[p]).wait()
            pltpu.make_async_copy(pos_hbm.at[pidx[p]], posb[p], sp[p]).wait()

            @plsc.parallel_loop(0, CH * NV, unroll=8)
            def _(i, p=p):
                r = i >> NV_SHIFT
                sl = pl.ds((i & (NV - 1)) * _LANES, _LANES)
                rows[p][r, sl] = rows[p][r, sl] + posb[p][r, sl]

            pltpu.async_copy(rows[p], out_hbm.at[oidx[p]], so[p])

            nxt = ci + NB - 1
            if nxt < NCH:
                q = nxt % NB
                if nxt >= NB:
                    pltpu.make_async_copy(
                        rows[q], out_hbm.at[oidx[q]], so[q]).wait()
                set_indices(q, nxt)
                issue_gathers(q)

        for m in range(NCH - NB, NCH):
            pltpu.make_async_copy(
                rows[m % NB], out_hbm.at[oidx[m % NB]], so[m % NB]).wait()

        @pl.when(wid == 0)
        def _():
            pltpu.make_async_copy(ga_v, out_hbm.at[pl.ds(0, B)], sg).wait()

    out_t_flat = pl.kernel(
        body,
        out_type=jax.ShapeDtypeStruct(((R + 1) * B, D), jnp.float32),
        mesh=mesh,
        scratch_types=[
            pltpu.VMEM((RW,), jnp.int32),
            pltpu.VMEM((CH,), jnp.int32),
            pltpu.VMEM((CH,), jnp.int32),
            pltpu.VMEM((CH,), jnp.int32),
            pltpu.VMEM((CH,), jnp.int32),
            pltpu.VMEM((CH,), jnp.int32),
            pltpu.VMEM((CH,), jnp.int32),
            pltpu.VMEM((CH,), jnp.int32),
            pltpu.VMEM((CH,), jnp.int32),
            pltpu.VMEM((CH,), jnp.int32),
            pltpu.VMEM((CH, D), jnp.float32),
            pltpu.VMEM((CH, D), jnp.float32),
            pltpu.VMEM((CH, D), jnp.float32),
            pltpu.VMEM((CH, D), jnp.float32),
            pltpu.VMEM((CH, D), jnp.float32),
            pltpu.VMEM((CH, D), jnp.float32),
            pltpu.VMEM((B, D), jnp.float32),
            pltpu.VMEM((1, D), jnp.float32),
            pltpu.SemaphoreType.DMA,
            pltpu.SemaphoreType.DMA,
            pltpu.SemaphoreType.DMA,
            pltpu.SemaphoreType.DMA,
            pltpu.SemaphoreType.DMA,
            pltpu.SemaphoreType.DMA,
            pltpu.SemaphoreType.DMA,
            pltpu.SemaphoreType.DMA,
            pltpu.SemaphoreType.DMA,
            pltpu.SemaphoreType.DMA,
        ],
    )(data_flat, idx_flat, pos, global_token)
    return out_t_flat


def _ridx_tc_kernel(idx_ref, out_ref):
    b = pl.program_id(0)
    row = idx_ref[b, :]
    out_ref[...] = jnp.broadcast_to(row[None, :, None], out_ref.shape)


@functools.partial(jax.jit, static_argnames=("D",))
def _ridx_run(remain_idx, *, D):
    B, R = remain_idx.shape
    return pl.pallas_call(
        _ridx_tc_kernel,
        grid=(B,),
        in_specs=[pl.BlockSpec((B, R), lambda b: (0, 0))],
        out_specs=pl.BlockSpec((1, R, D), lambda b: (b, 0, 0)),
        out_shape=jax.ShapeDtypeStruct((B, R, D), jnp.int32),
    )(remain_idx)


def kernel(data, remain_idx, global_token):
    B, S, D = data.shape
    R = remain_idx.shape[1]
    pos = jnp.asarray(_positional_encoding_np(D)[: S + 1, :])
    out_t_flat = _run(
        data.reshape(B * S, D),
        remain_idx.reshape(B * R),
        pos,
        global_token,
        B=B, S=S, R=R, D=D,
    )
    ridx = _ridx_run(remain_idx, D=D)
    out = out_t_flat.reshape(R + 1, B, D).transpose(1, 0, 2)
    return (out, ridx)
